# baseline (speedup 1.0000x reference)
import functools

import jax
import jax.numpy as jnp
import numpy as np
from jax import lax
from jax.experimental import pallas as pl
from jax.experimental.pallas import tpu as pltpu
from jax.experimental.pallas import tpu_sc as plsc

NC = 2
NS = 16
CH = 128
CHD = 64
GSD = 32


def _stage_a(xp, W_lin, W_root, regp, region_emb, bias2, NP, D, NREG):
    BM = 1024

    def body(x_ref, wl_ref, wr_ref, reg_ref, emb_ref, b_ref, xw_ref, pre_ref):
        xb = x_ref[...]
        xw_ref[...] = jnp.dot(
            xb, wl_ref[...], preferred_element_type=jnp.float32
        ).astype(jnp.bfloat16)
        oh = (reg_ref[...] == lax.broadcasted_iota(jnp.int32, (BM, NREG), 1))
        pre_ref[...] = (
            jnp.dot(xb, wr_ref[...], preferred_element_type=jnp.float32)
            + jnp.dot(oh.astype(jnp.float32), emb_ref[...],
                      preferred_element_type=jnp.float32)
            + b_ref[...]
        )

    return pl.pallas_call(
        body,
        grid=(NP // BM,),
        in_specs=[
            pl.BlockSpec((BM, D), lambda i: (i, 0)),
            pl.BlockSpec((D, D), lambda i: (0, 0)),
            pl.BlockSpec((D, D), lambda i: (0, 0)),
            pl.BlockSpec((BM, 1), lambda i: (i, 0)),
            pl.BlockSpec((NREG, D), lambda i: (0, 0)),
            pl.BlockSpec((1, D), lambda i: (0, 0)),
        ],
        out_specs=[pl.BlockSpec((BM, D), lambda i: (i, 0))] * 2,
        out_shape=[jax.ShapeDtypeStruct((NP, D), jnp.bfloat16),
                   jax.ShapeDtypeStruct((NP, D), jnp.float32)],
    )(xp, W_lin, W_root, regp, region_emb, bias2)


def _stage_b(pos8, src2, dst2, NP, EP, CPW):
    mesh = plsc.VectorSubcoreMesh(core_axis_name="c", subcore_axis_name="s")
    slab = NP // NS

    @functools.partial(
        pl.kernel,
        out_type=(
            jax.ShapeDtypeStruct((EP, 16), jnp.float32),
            jax.ShapeDtypeStruct((NP, 16), jnp.float32),
            jax.ShapeDtypeStruct((NP, 16), jnp.float32),
        ),
        mesh=mesh,
        scratch_types=[
            pltpu.VMEM((CPW, CH), jnp.int32),
            pltpu.VMEM((CPW, CH), jnp.int32),
            pltpu.VMEM((2, CH, 16), jnp.float32),
            pltpu.VMEM((2, CH, 16), jnp.float32),
            pltpu.VMEM((CH, 16), jnp.float32),
            pltpu.VMEM_SHARED((NP, 16), jnp.float32),
            pltpu.SemaphoreType.DMA((2,)),
            pltpu.SemaphoreType.DMA((2,)),
        ],
        compiler_params=pltpu.CompilerParams(use_tc_tiling_on_sc=False,
                                            needs_layout_passes=False),
    )
    def body(pos8_h, src_h, dst_h, rel8_h, deg0_h, deg1_h,
             sidx2, didx2, ps2, pd2, ones_v, deg_sh, sem_s, sem_d):
        c = lax.axis_index("c")
        s = lax.axis_index("s")
        wid = c * NS + s
        pltpu.sync_copy(src_h.at[pl.ds(wid * CPW, CPW)], sidx2)
        pltpu.sync_copy(dst_h.at[pl.ds(wid * CPW, CPW)], didx2)

        def _fill(i, carry):
            ones_v[i, :] = jnp.ones((16,), jnp.float32)
            ps2[0, i, :] = jnp.zeros((16,), jnp.float32)
            return carry

        lax.fori_loop(0, CH, _fill, 0)

        def _zero(b, carry):
            pltpu.sync_copy(ps2.at[0], deg_sh.at[pl.ds(s * slab + b * CH, CH)])
            return carry

        lax.fori_loop(0, slab // CH, _zero, 0)
        plsc.subcore_barrier()

        def _issue(j):
            b = lax.rem(j, 2)
            pltpu.async_copy(pos8_h.at[sidx2.at[j]], ps2.at[b], sem_s.at[b])
            pltpu.async_copy(pos8_h.at[didx2.at[j]], pd2.at[b], sem_d.at[b])

        _issue(0)

        def _chunk(j, carry):
            b = lax.rem(j, 2)
            base = (wid * CPW + j) * CH

            @pl.when(j + 1 < CPW)
            def _():
                _issue(j + 1)

            pltpu.make_async_copy(pos8_h.at[sidx2.at[j]], ps2.at[b],
                                  sem_s.at[b]).wait()
            pltpu.make_async_copy(pos8_h.at[didx2.at[j]], pd2.at[b],
                                  sem_d.at[b]).wait()

            def _sub(r, carry2):
                ps2[b, r, :] = ps2[b, r, :] - pd2[b, r, :]
                return carry2

            lax.fori_loop(0, CH, _sub, 0)
            pltpu.sync_copy(ps2.at[b], rel8_h.at[pl.ds(base, CH)])
            pltpu.sync_copy(ones_v, deg_sh.at[didx2.at[j]], add=True)
            return carry

        lax.fori_loop(0, CPW, _chunk, 0)
        plsc.subcore_barrier()

        @pl.when(c == 0)
        def _():
            pltpu.sync_copy(deg_sh.at[pl.ds(s * slab, slab)],
                            deg0_h.at[pl.ds(s * slab, slab)])

        @pl.when(c == 1)
        def _():
            pltpu.sync_copy(deg_sh.at[pl.ds(s * slab, slab)],
                            deg1_h.at[pl.ds(s * slab, slab)])

    return body(pos8, src2, dst2)


def _stage_c(rel8, w1x, w1y, b1r, W2, b2r, EP, D):
    BE = 2048

    def body(rel_ref, w1x_ref, w1y_ref, b1_ref, w2_ref, b2_ref, ew_ref):
        rel = rel_ref[...]
        h = jnp.maximum(
            rel[:, 0:1] * w1x_ref[...] + rel[:, 1:2] * w1y_ref[...] + b1_ref[...],
            0.0)
        ew_ref[...] = (
            jnp.dot(h.astype(jnp.bfloat16), w2_ref[...],
                    preferred_element_type=jnp.float32)
            + b2_ref[...]
        ).astype(jnp.bfloat16)

    return pl.pallas_call(
        body,
        grid=(EP // BE,),
        in_specs=[
            pl.BlockSpec((BE, 16), lambda i: (i, 0)),
            pl.BlockSpec((1, D), lambda i: (0, 0)),
            pl.BlockSpec((1, D), lambda i: (0, 0)),
            pl.BlockSpec((1, D), lambda i: (0, 0)),
            pl.BlockSpec((D, D), lambda i: (0, 0)),
            pl.BlockSpec((1, D), lambda i: (0, 0)),
        ],
        out_specs=pl.BlockSpec((BE, D), lambda i: (i, 0)),
        out_shape=jax.ShapeDtypeStruct((EP, D), jnp.bfloat16),
    )(rel8, w1x, w1y, b1r, W2, b2r)


def _stage_d(xw, src2, dst2, ew, NP, EP, CPW, D):
    mesh = plsc.VectorSubcoreMesh(core_axis_name="c", subcore_axis_name="s")
    slab = NP // NS
    GS = GSD
    NG = CPW // GS

    @functools.partial(
        pl.kernel,
        out_type=(
            jax.ShapeDtypeStruct((NP, D), jnp.float32),
            jax.ShapeDtypeStruct((NP, D), jnp.float32),
        ),
        mesh=mesh,
        scratch_types=[
            pltpu.VMEM((2, GS, CHD), jnp.int32),
            pltpu.VMEM((2, GS, CHD), jnp.int32),
            pltpu.VMEM((2, CHD, D), jnp.bfloat16),
            pltpu.VMEM((2, CHD, D), jnp.bfloat16),
            pltpu.VMEM((CHD, D), jnp.float32),
            pltpu.VMEM_SHARED((NP, D), jnp.float32),
            pltpu.SemaphoreType.DMA((2,)),
            pltpu.SemaphoreType.DMA((2,)),
            pltpu.SemaphoreType.DMA((2,)),
            pltpu.SemaphoreType.DMA((2,)),
        ],
        compiler_params=pltpu.CompilerParams(use_tc_tiling_on_sc=False,
                                            needs_layout_passes=False),
    )
    def body(xw_h, src_h, dst_h, ew_h, agg0_h, agg1_h,
             sidx2, didx2, xs2, ew2, msg, agg_sh, gsem, esem, ssem, dsem):
        c = lax.axis_index("c")
        s = lax.axis_index("s")
        wid = c * NS + s

        def _zr(r, carry):
            for k in range(D // 16):
                msg[r, pl.ds(k * 16, 16)] = jnp.zeros((16,), jnp.float32)
            return carry

        lax.fori_loop(0, CHD, _zr, 0)

        def _zero(b, carry):
            pltpu.sync_copy(msg, agg_sh.at[pl.ds(s * slab + b * CHD, CHD)])
            return carry

        lax.fori_loop(0, slab // CHD, _zero, 0)
        plsc.subcore_barrier()

        def _issue_idx(g):
            bg = lax.rem(g, 2)
            row = wid * CPW + g * GS
            pltpu.async_copy(src_h.at[pl.ds(row, GS)], sidx2.at[bg], ssem.at[bg])
            pltpu.async_copy(dst_h.at[pl.ds(row, GS)], didx2.at[bg], dsem.at[bg])

        def _wait_idx(g):
            bg = lax.rem(g, 2)
            row = wid * CPW + g * GS
            pltpu.make_async_copy(src_h.at[pl.ds(row, GS)], sidx2.at[bg],
                                  ssem.at[bg]).wait()
            pltpu.make_async_copy(dst_h.at[pl.ds(row, GS)], didx2.at[bg],
                                  dsem.at[bg]).wait()

        def _issue_gather(j):
            b = lax.rem(j, 2)
            g = lax.div(j, GS)
            bg = lax.rem(g, 2)
            jj = lax.rem(j, GS)
            base = (wid * CPW + j) * CHD
            pltpu.async_copy(xw_h.at[sidx2.at[bg, jj]], xs2.at[b], gsem.at[b])
            pltpu.async_copy(ew_h.at[pl.ds(base, CHD)], ew2.at[b], esem.at[b])

        _issue_idx(0)
        _wait_idx(0)
        _issue_gather(0)

        def _chunk(j, carry):
            b = lax.rem(j, 2)
            g = lax.div(j, GS)
            bg = lax.rem(g, 2)
            jj = lax.rem(j, GS)
            base = (wid * CPW + j) * CHD

            @pl.when(jnp.logical_and(jj == 0, g + 1 < NG))
            def _():
                _issue_idx(g + 1)

            @pl.when(j + 1 < CPW)
            def _():
                @pl.when(lax.rem(j + 1, GS) == 0)
                def _():
                    _wait_idx(g + 1)

                _issue_gather(j + 1)

            pltpu.make_async_copy(xw_h.at[sidx2.at[bg, jj]], xs2.at[b],
                                  gsem.at[b]).wait()
            pltpu.make_async_copy(ew_h.at[pl.ds(base, CHD)], ew2.at[b],
                                  esem.at[b]).wait()

            def _mul(r, carry2):
                for k in range(D // 32):
                    sl32 = pl.ds(k * 32, 32)
                    xlo, xhi = plsc.unpack(xs2[b, r, sl32],
                                           format=plsc.PackFormat.INTERLEAVED)
                    elo, ehi = plsc.unpack(ew2[b, r, sl32],
                                           format=plsc.PackFormat.INTERLEAVED)
                    msg[r, pl.ds(k * 32, 16)] = xlo * elo
                    msg[r, pl.ds(k * 32 + 16, 16)] = xhi * ehi
                return carry2

            lax.fori_loop(0, CHD, _mul, 0)
            pltpu.sync_copy(msg, agg_sh.at[didx2.at[bg, jj]], add=True)
            return carry

        lax.fori_loop(0, CPW, _chunk, 0)
        plsc.subcore_barrier()

        @pl.when(c == 0)
        def _():
            pltpu.sync_copy(agg_sh.at[pl.ds(s * slab, slab)],
                            agg0_h.at[pl.ds(s * slab, slab)])

        @pl.when(c == 1)
        def _():
            pltpu.sync_copy(agg_sh.at[pl.ds(s * slab, slab)],
                            agg1_h.at[pl.ds(s * slab, slab)])

    return body(xw, src2, dst2, ew)


def _stage_e(pre, agg0, agg1, deg0, deg1, NP, D):
    BM = 1024

    def body(pre_ref, a0_ref, a1_ref, d0_ref, d1_ref, out_ref):
        deg = d0_ref[...][:, 0:1] + d1_ref[...][:, 0:1]
        r = 1.0 / jnp.maximum(deg, 1.0)
        out_ref[...] = jnp.maximum(
            pre_ref[...] + (a0_ref[...] + a1_ref[...]) * r, 0.0)

    return pl.pallas_call(
        body,
        grid=(NP // BM,),
        in_specs=[
            pl.BlockSpec((BM, D), lambda i: (i, 0)),
            pl.BlockSpec((BM, D), lambda i: (i, 0)),
            pl.BlockSpec((BM, D), lambda i: (i, 0)),
            pl.BlockSpec((BM, 16), lambda i: (i, 0)),
            pl.BlockSpec((BM, 16), lambda i: (i, 0)),
        ],
        out_specs=pl.BlockSpec((BM, D), lambda i: (i, 0)),
        out_shape=jax.ShapeDtypeStruct((NP, D), jnp.float32),
    )(pre, agg0, agg1, deg0, deg1)


def kernel(x, edge_index, pos, region, W_lin, W1, b1, W2, b2, W_root, bias,
           region_emb):
    N, D = x.shape
    E = edge_index.shape[1]
    NREG = region_emb.shape[0]

    NP = -(-(N + 1) // (NS * CH)) * (NS * CH)
    CPWD = -(-E // (NC * NS * CHD))
    CPWD = -(-CPWD // GSD) * GSD
    EP = NC * NS * CHD * CPWD
    CPWB = EP // (NC * NS * CH)

    srcf = jnp.pad(edge_index[0].astype(jnp.int32), (0, EP - E))
    dstf = jnp.pad(edge_index[1].astype(jnp.int32), (0, EP - E),
                   constant_values=N)
    pos8 = jnp.pad(pos, ((0, NP - N), (0, 16 - pos.shape[1])))
    xp = jnp.pad(x, ((0, NP - N), (0, 0)))
    regp = jnp.pad(region.astype(jnp.int32), (0, NP - N)).reshape(NP, 1)
    bias2 = bias.reshape(1, D)

    perm = np.empty((D,), dtype=np.int32)
    for g in range(D // 32):
        for i in range(16):
            perm[g * 32 + 2 * i] = g * 32 + i
            perm[g * 32 + 2 * i + 1] = g * 32 + 16 + i
    W_lin_p = W_lin[:, jnp.asarray(perm)]

    xw, pre = _stage_a(xp, W_lin_p, W_root, regp, region_emb, bias2, NP, D,
                       NREG)
    rel8, deg0, deg1 = _stage_b(pos8, srcf.reshape(EP // CH, CH),
                                dstf.reshape(EP // CH, CH), NP, EP, CPWB)
    ew = _stage_c(rel8,
                  W1[0].reshape(1, D), W1[1].reshape(1, D), b1.reshape(1, D),
                  W2[:, jnp.asarray(perm)].astype(jnp.bfloat16),
                  b2[jnp.asarray(perm)].reshape(1, D), EP, D)
    agg0, agg1 = _stage_d(xw, srcf.reshape(EP // CHD, CHD),
                          dstf.reshape(EP // CHD, CHD), ew, NP, EP, CPWD, D)
    outp = _stage_e(pre, agg0, agg1, deg0, deg1, NP, D)
    return outp[:N]


# speedup vs baseline: 1.1248x; 1.1248x over previous
import functools

import jax
import jax.numpy as jnp
import numpy as np
from jax import lax
from jax.experimental import pallas as pl
from jax.experimental.pallas import tpu as pltpu
from jax.experimental.pallas import tpu_sc as plsc

NC = 2
NS = 16
CH = 128
CHD = 64
GSD = 32


def _pack_pairs(lo_bf, hi_bf):
    lo = lax.bitcast_convert_type(lo_bf, jnp.uint16).astype(jnp.uint32)
    hi = lax.bitcast_convert_type(hi_bf, jnp.uint16).astype(jnp.uint32)
    return lax.bitcast_convert_type(lo | (hi << 16), jnp.float32)


def _stage_a(xp, W_lo, W_hi, W_root, regp, region_emb, bias2, NP, D, NREG):
    BM = 1024

    def body(x_ref, wlo_ref, whi_ref, wr_ref, reg_ref, emb_ref, b_ref,
             xw_ref, pre_ref):
        xb = x_ref[...]
        lo = jnp.dot(xb, wlo_ref[...],
                     preferred_element_type=jnp.float32).astype(jnp.bfloat16)
        hi = jnp.dot(xb, whi_ref[...],
                     preferred_element_type=jnp.float32).astype(jnp.bfloat16)
        xw_ref[...] = _pack_pairs(lo, hi)
        oh = (reg_ref[...] == lax.broadcasted_iota(jnp.int32, (BM, NREG), 1))
        pre_ref[...] = (
            jnp.dot(xb, wr_ref[...], preferred_element_type=jnp.float32)
            + jnp.dot(oh.astype(jnp.float32), emb_ref[...],
                      preferred_element_type=jnp.float32)
            + b_ref[...]
        )

    return pl.pallas_call(
        body,
        grid=(NP // BM,),
        in_specs=[
            pl.BlockSpec((BM, D), lambda i: (i, 0)),
            pl.BlockSpec((D, D // 2), lambda i: (0, 0)),
            pl.BlockSpec((D, D // 2), lambda i: (0, 0)),
            pl.BlockSpec((D, D), lambda i: (0, 0)),
            pl.BlockSpec((BM, 1), lambda i: (i, 0)),
            pl.BlockSpec((NREG, D), lambda i: (0, 0)),
            pl.BlockSpec((1, D), lambda i: (0, 0)),
        ],
        out_specs=[pl.BlockSpec((BM, D // 2), lambda i: (i, 0)),
                   pl.BlockSpec((BM, D), lambda i: (i, 0))],
        out_shape=[jax.ShapeDtypeStruct((NP, D // 2), jnp.float32),
                   jax.ShapeDtypeStruct((NP, D), jnp.float32)],
    )(xp, W_lo, W_hi, W_root, regp, region_emb, bias2)


def _stage_b(pos8, src2, dst2, NP, EP, CPW):
    mesh = plsc.VectorSubcoreMesh(core_axis_name="c", subcore_axis_name="s")
    slab = NP // NS

    @functools.partial(
        pl.kernel,
        out_type=(
            jax.ShapeDtypeStruct((EP, 16), jnp.float32),
            jax.ShapeDtypeStruct((NP, 16), jnp.float32),
            jax.ShapeDtypeStruct((NP, 16), jnp.float32),
        ),
        mesh=mesh,
        scratch_types=[
            pltpu.VMEM((CPW, CH), jnp.int32),
            pltpu.VMEM((CPW, CH), jnp.int32),
            pltpu.VMEM((2, CH, 16), jnp.float32),
            pltpu.VMEM((2, CH, 16), jnp.float32),
            pltpu.VMEM((CH, 16), jnp.float32),
            pltpu.VMEM_SHARED((NP, 16), jnp.float32),
            pltpu.SemaphoreType.DMA((2,)),
            pltpu.SemaphoreType.DMA((2,)),
        ],
        compiler_params=pltpu.CompilerParams(use_tc_tiling_on_sc=False,
                                            needs_layout_passes=False),
    )
    def body(pos8_h, src_h, dst_h, rel8_h, deg0_h, deg1_h,
             sidx2, didx2, ps2, pd2, ones_v, deg_sh, sem_s, sem_d):
        c = lax.axis_index("c")
        s = lax.axis_index("s")
        wid = c * NS + s
        pltpu.sync_copy(src_h.at[pl.ds(wid * CPW, CPW)], sidx2)
        pltpu.sync_copy(dst_h.at[pl.ds(wid * CPW, CPW)], didx2)

        def _fill(i, carry):
            ones_v[i, :] = jnp.ones((16,), jnp.float32)
            ps2[0, i, :] = jnp.zeros((16,), jnp.float32)
            return carry

        lax.fori_loop(0, CH, _fill, 0)

        def _zero(b, carry):
            pltpu.sync_copy(ps2.at[0], deg_sh.at[pl.ds(s * slab + b * CH, CH)])
            return carry

        lax.fori_loop(0, slab // CH, _zero, 0)
        plsc.subcore_barrier()

        def _issue(j):
            b = lax.rem(j, 2)
            pltpu.async_copy(pos8_h.at[sidx2.at[j]], ps2.at[b], sem_s.at[b])
            pltpu.async_copy(pos8_h.at[didx2.at[j]], pd2.at[b], sem_d.at[b])

        _issue(0)

        def _chunk(j, carry):
            b = lax.rem(j, 2)
            base = (wid * CPW + j) * CH

            @pl.when(j + 1 < CPW)
            def _():
                _issue(j + 1)

            pltpu.make_async_copy(pos8_h.at[sidx2.at[j]], ps2.at[b],
                                  sem_s.at[b]).wait()
            pltpu.make_async_copy(pos8_h.at[didx2.at[j]], pd2.at[b],
                                  sem_d.at[b]).wait()

            def _sub(r, carry2):
                ps2[b, r, :] = ps2[b, r, :] - pd2[b, r, :]
                return carry2

            lax.fori_loop(0, CH, _sub, 0)
            pltpu.sync_copy(ps2.at[b], rel8_h.at[pl.ds(base, CH)])
            pltpu.sync_copy(ones_v, deg_sh.at[didx2.at[j]], add=True)
            return carry

        lax.fori_loop(0, CPW, _chunk, 0)
        plsc.subcore_barrier()

        @pl.when(c == 0)
        def _():
            pltpu.sync_copy(deg_sh.at[pl.ds(s * slab, slab)],
                            deg0_h.at[pl.ds(s * slab, slab)])

        @pl.when(c == 1)
        def _():
            pltpu.sync_copy(deg_sh.at[pl.ds(s * slab, slab)],
                            deg1_h.at[pl.ds(s * slab, slab)])

    return body(pos8, src2, dst2)


def _stage_c(rel8, w1x, w1y, b1r, W2lo, W2hi, b2lo, b2hi, EP, D):
    BE = 2048

    def body(rel_ref, w1x_ref, w1y_ref, b1_ref, w2lo_ref, w2hi_ref,
             b2lo_ref, b2hi_ref, ew_ref):
        rel = rel_ref[...]
        h = jnp.maximum(
            rel[:, 0:1] * w1x_ref[...] + rel[:, 1:2] * w1y_ref[...] + b1_ref[...],
            0.0).astype(jnp.bfloat16)
        lo = (jnp.dot(h, w2lo_ref[...], preferred_element_type=jnp.float32)
              + b2lo_ref[...]).astype(jnp.bfloat16)
        hi = (jnp.dot(h, w2hi_ref[...], preferred_element_type=jnp.float32)
              + b2hi_ref[...]).astype(jnp.bfloat16)
        ew_ref[...] = _pack_pairs(lo, hi)

    return pl.pallas_call(
        body,
        grid=(EP // BE,),
        in_specs=[
            pl.BlockSpec((BE, 16), lambda i: (i, 0)),
            pl.BlockSpec((1, D), lambda i: (0, 0)),
            pl.BlockSpec((1, D), lambda i: (0, 0)),
            pl.BlockSpec((1, D), lambda i: (0, 0)),
            pl.BlockSpec((D, D // 2), lambda i: (0, 0)),
            pl.BlockSpec((D, D // 2), lambda i: (0, 0)),
            pl.BlockSpec((1, D // 2), lambda i: (0, 0)),
            pl.BlockSpec((1, D // 2), lambda i: (0, 0)),
        ],
        out_specs=pl.BlockSpec((BE, D // 2), lambda i: (i, 0)),
        out_shape=jax.ShapeDtypeStruct((EP, D // 2), jnp.float32),
    )(rel8, w1x, w1y, b1r, W2lo, W2hi, b2lo, b2hi)


def _stage_d(xw, src2, dst2, ew, NP, EP, CPW, D):
    mesh = plsc.VectorSubcoreMesh(core_axis_name="c", subcore_axis_name="s")
    slab = NP // NS
    GS = GSD
    NG = CPW // GS

    @functools.partial(
        pl.kernel,
        out_type=(
            jax.ShapeDtypeStruct((NP, D), jnp.float32),
            jax.ShapeDtypeStruct((NP, D), jnp.float32),
        ),
        mesh=mesh,
        scratch_types=[
            pltpu.VMEM((2, GS, CHD), jnp.int32),
            pltpu.VMEM((2, GS, CHD), jnp.int32),
            pltpu.VMEM((2, CHD, D // 2), jnp.float32),
            pltpu.VMEM((2, CHD, D // 2), jnp.float32),
            pltpu.VMEM((CHD, D), jnp.float32),
            pltpu.VMEM_SHARED((NP, D), jnp.float32),
            pltpu.SemaphoreType.DMA((2,)),
            pltpu.SemaphoreType.DMA((2,)),
            pltpu.SemaphoreType.DMA((2,)),
            pltpu.SemaphoreType.DMA((2,)),
        ],
        compiler_params=pltpu.CompilerParams(use_tc_tiling_on_sc=False,
                                            needs_layout_passes=False),
    )
    def body(xw_h, src_h, dst_h, ew_h, agg0_h, agg1_h,
             sidx2, didx2, xs2, ew2, msg, agg_sh, gsem, esem, ssem, dsem):
        c = lax.axis_index("c")
        s = lax.axis_index("s")
        wid = c * NS + s

        def _zr(r, carry):
            for k in range(D // 16):
                msg[r, pl.ds(k * 16, 16)] = jnp.zeros((16,), jnp.float32)
            return carry

        lax.fori_loop(0, CHD, _zr, 0)

        def _zero(b, carry):
            pltpu.sync_copy(msg, agg_sh.at[pl.ds(s * slab + b * CHD, CHD)])
            return carry

        lax.fori_loop(0, slab // CHD, _zero, 0)
        plsc.subcore_barrier()

        def _issue_idx(g):
            bg = lax.rem(g, 2)
            row = wid * CPW + g * GS
            pltpu.async_copy(src_h.at[pl.ds(row, GS)], sidx2.at[bg], ssem.at[bg])
            pltpu.async_copy(dst_h.at[pl.ds(row, GS)], didx2.at[bg], dsem.at[bg])

        def _wait_idx(g):
            bg = lax.rem(g, 2)
            row = wid * CPW + g * GS
            pltpu.make_async_copy(src_h.at[pl.ds(row, GS)], sidx2.at[bg],
                                  ssem.at[bg]).wait()
            pltpu.make_async_copy(dst_h.at[pl.ds(row, GS)], didx2.at[bg],
                                  dsem.at[bg]).wait()

        def _issue_gather(j):
            b = lax.rem(j, 2)
            g = lax.div(j, GS)
            bg = lax.rem(g, 2)
            jj = lax.rem(j, GS)
            base = (wid * CPW + j) * CHD
            pltpu.async_copy(xw_h.at[sidx2.at[bg, jj]], xs2.at[b], gsem.at[b])
            pltpu.async_copy(ew_h.at[pl.ds(base, CHD)], ew2.at[b], esem.at[b])

        _issue_idx(0)
        _wait_idx(0)
        _issue_gather(0)

        def _chunk(j, carry):
            b = lax.rem(j, 2)
            g = lax.div(j, GS)
            bg = lax.rem(g, 2)
            jj = lax.rem(j, GS)
            base = (wid * CPW + j) * CHD

            @pl.when(jnp.logical_and(jj == 0, g + 1 < NG))
            def _():
                _issue_idx(g + 1)

            @pl.when(j + 1 < CPW)
            def _():
                @pl.when(lax.rem(j + 1, GS) == 0)
                def _():
                    _wait_idx(g + 1)

                _issue_gather(j + 1)

            pltpu.make_async_copy(xw_h.at[sidx2.at[bg, jj]], xs2.at[b],
                                  gsem.at[b]).wait()
            pltpu.make_async_copy(ew_h.at[pl.ds(base, CHD)], ew2.at[b],
                                  esem.at[b]).wait()

            def _mul(r, carry2):
                for k in range(D // 32):
                    slw = pl.ds(k * 16, 16)
                    xbf = plsc.bitcast(xs2[b, r, slw], jnp.bfloat16)
                    ebf = plsc.bitcast(ew2[b, r, slw], jnp.bfloat16)
                    xlo, xhi = plsc.unpack(xbf,
                                           format=plsc.PackFormat.INTERLEAVED)
                    elo, ehi = plsc.unpack(ebf,
                                           format=plsc.PackFormat.INTERLEAVED)
                    msg[r, pl.ds(k * 32, 16)] = xlo * elo
                    msg[r, pl.ds(k * 32 + 16, 16)] = xhi * ehi
                return carry2

            lax.fori_loop(0, CHD, _mul, 0)
            pltpu.sync_copy(msg, agg_sh.at[didx2.at[bg, jj]], add=True)
            return carry

        lax.fori_loop(0, CPW, _chunk, 0)
        plsc.subcore_barrier()

        @pl.when(c == 0)
        def _():
            pltpu.sync_copy(agg_sh.at[pl.ds(s * slab, slab)],
                            agg0_h.at[pl.ds(s * slab, slab)])

        @pl.when(c == 1)
        def _():
            pltpu.sync_copy(agg_sh.at[pl.ds(s * slab, slab)],
                            agg1_h.at[pl.ds(s * slab, slab)])

    return body(xw, src2, dst2, ew)


def _stage_e(pre, agg0, agg1, deg0, deg1, NP, D):
    BM = 1024

    def body(pre_ref, a0_ref, a1_ref, d0_ref, d1_ref, out_ref):
        deg = d0_ref[...][:, 0:1] + d1_ref[...][:, 0:1]
        r = 1.0 / jnp.maximum(deg, 1.0)
        out_ref[...] = jnp.maximum(
            pre_ref[...] + (a0_ref[...] + a1_ref[...]) * r, 0.0)

    return pl.pallas_call(
        body,
        grid=(NP // BM,),
        in_specs=[
            pl.BlockSpec((BM, D), lambda i: (i, 0)),
            pl.BlockSpec((BM, D), lambda i: (i, 0)),
            pl.BlockSpec((BM, D), lambda i: (i, 0)),
            pl.BlockSpec((BM, 16), lambda i: (i, 0)),
            pl.BlockSpec((BM, 16), lambda i: (i, 0)),
        ],
        out_specs=pl.BlockSpec((BM, D), lambda i: (i, 0)),
        out_shape=jax.ShapeDtypeStruct((NP, D), jnp.float32),
    )(pre, agg0, agg1, deg0, deg1)


def kernel(x, edge_index, pos, region, W_lin, W1, b1, W2, b2, W_root, bias,
           region_emb):
    N, D = x.shape
    E = edge_index.shape[1]
    NREG = region_emb.shape[0]

    NP = -(-(N + 1) // (NS * CH)) * (NS * CH)
    CPWD = -(-E // (NC * NS * CHD))
    CPWD = -(-CPWD // GSD) * GSD
    EP = NC * NS * CHD * CPWD
    CPWB = EP // (NC * NS * CH)

    srcf = jnp.pad(edge_index[0].astype(jnp.int32), (0, EP - E))
    dstf = jnp.pad(edge_index[1].astype(jnp.int32), (0, EP - E),
                   constant_values=N)
    pos8 = jnp.pad(pos, ((0, NP - N), (0, 16 - pos.shape[1])))
    xp = jnp.pad(x, ((0, NP - N), (0, 0)))
    regp = jnp.pad(region.astype(jnp.int32), (0, NP - N)).reshape(NP, 1)
    bias2 = bias.reshape(1, D)

    lo_idx = np.concatenate([np.arange(g * 32, g * 32 + 16)
                             for g in range(D // 32)])
    hi_idx = lo_idx + 16
    W_lo = W_lin[:, jnp.asarray(lo_idx)]
    W_hi = W_lin[:, jnp.asarray(hi_idx)]

    xw, pre = _stage_a(xp, W_lo, W_hi, W_root, regp, region_emb, bias2, NP, D,
                       NREG)
    rel8, deg0, deg1 = _stage_b(pos8, srcf.reshape(EP // CH, CH),
                                dstf.reshape(EP // CH, CH), NP, EP, CPWB)
    ew = _stage_c(rel8,
                  W1[0].reshape(1, D), W1[1].reshape(1, D), b1.reshape(1, D),
                  W2[:, jnp.asarray(lo_idx)].astype(jnp.bfloat16),
                  W2[:, jnp.asarray(hi_idx)].astype(jnp.bfloat16),
                  b2[jnp.asarray(lo_idx)].reshape(1, D // 2),
                  b2[jnp.asarray(hi_idx)].reshape(1, D // 2), EP, D)
    agg0, agg1 = _stage_d(xw, srcf.reshape(EP // CHD, CHD),
                          dstf.reshape(EP // CHD, CHD), ew, NP, EP, CPWD, D)
    outp = _stage_e(pre, agg0, agg1, deg0, deg1, NP, D)
    return outp[:N]


# speedup vs baseline: 1.2983x; 1.1543x over previous
import functools

import jax
import jax.numpy as jnp
import numpy as np
from jax import lax
from jax.experimental import pallas as pl
from jax.experimental.pallas import tpu as pltpu
from jax.experimental.pallas import tpu_sc as plsc

NC = 2
NS = 16
CH = 128
CHD = 64
GSD = 32


def _pack_pairs(lo_bf, hi_bf):
    lo = lax.bitcast_convert_type(lo_bf, jnp.uint16).astype(jnp.uint32)
    hi = lax.bitcast_convert_type(hi_bf, jnp.uint16).astype(jnp.uint32)
    return lax.bitcast_convert_type(lo | (hi << 16), jnp.float32)


def _stage_a(xp, W_lo, W_hi, W_root, regp, region_emb, bias2, NP, D, NREG):
    BM = 1024

    def body(x_ref, wlo_ref, whi_ref, wr_ref, reg_ref, emb_ref, b_ref,
             xw_ref, pre_ref):
        xb = x_ref[...]
        lo = jnp.dot(xb, wlo_ref[...],
                     preferred_element_type=jnp.float32).astype(jnp.bfloat16)
        hi = jnp.dot(xb, whi_ref[...],
                     preferred_element_type=jnp.float32).astype(jnp.bfloat16)
        xw_ref[...] = _pack_pairs(lo, hi)
        oh = (reg_ref[...] == lax.broadcasted_iota(jnp.int32, (BM, NREG), 1))
        pre_ref[...] = (
            jnp.dot(xb, wr_ref[...], preferred_element_type=jnp.float32)
            + jnp.dot(oh.astype(jnp.float32), emb_ref[...],
                      preferred_element_type=jnp.float32)
            + b_ref[...]
        )

    return pl.pallas_call(
        body,
        grid=(NP // BM,),
        in_specs=[
            pl.BlockSpec((BM, D), lambda i: (i, 0)),
            pl.BlockSpec((D, D // 2), lambda i: (0, 0)),
            pl.BlockSpec((D, D // 2), lambda i: (0, 0)),
            pl.BlockSpec((D, D), lambda i: (0, 0)),
            pl.BlockSpec((BM, 1), lambda i: (i, 0)),
            pl.BlockSpec((NREG, D), lambda i: (0, 0)),
            pl.BlockSpec((1, D), lambda i: (0, 0)),
        ],
        out_specs=[pl.BlockSpec((BM, D // 2), lambda i: (i, 0)),
                   pl.BlockSpec((BM, D), lambda i: (i, 0))],
        out_shape=[jax.ShapeDtypeStruct((NP, D // 2), jnp.float32),
                   jax.ShapeDtypeStruct((NP, D), jnp.float32)],
    )(xp, W_lo, W_hi, W_root, regp, region_emb, bias2)


def _stage_b(pos8, src2, dst2, NP, EP, CPW):
    mesh = plsc.VectorSubcoreMesh(core_axis_name="c", subcore_axis_name="s")
    slab = NP // NS

    @functools.partial(
        pl.kernel,
        out_type=(
            jax.ShapeDtypeStruct((EP, 16), jnp.float32),
            jax.ShapeDtypeStruct((NP, 16), jnp.float32),
            jax.ShapeDtypeStruct((NP, 16), jnp.float32),
        ),
        mesh=mesh,
        scratch_types=[
            pltpu.VMEM((CPW, CH), jnp.int32),
            pltpu.VMEM((CPW, CH), jnp.int32),
            pltpu.VMEM((2, CH, 16), jnp.float32),
            pltpu.VMEM((2, CH, 16), jnp.float32),
            pltpu.VMEM((CH, 16), jnp.float32),
            pltpu.VMEM_SHARED((NP, 16), jnp.float32),
            pltpu.SemaphoreType.DMA((2,)),
            pltpu.SemaphoreType.DMA((2,)),
        ],
        compiler_params=pltpu.CompilerParams(use_tc_tiling_on_sc=False,
                                            needs_layout_passes=False),
    )
    def body(pos8_h, src_h, dst_h, rel8_h, deg0_h, deg1_h,
             sidx2, didx2, ps2, pd2, ones_v, deg_sh, sem_s, sem_d):
        c = lax.axis_index("c")
        s = lax.axis_index("s")
        wid = c * NS + s
        pltpu.sync_copy(src_h.at[pl.ds(wid * CPW, CPW)], sidx2)
        pltpu.sync_copy(dst_h.at[pl.ds(wid * CPW, CPW)], didx2)

        def _fill(i, carry):
            ones_v[i, :] = jnp.ones((16,), jnp.float32)
            ps2[0, i, :] = jnp.zeros((16,), jnp.float32)
            return carry

        lax.fori_loop(0, CH, _fill, 0)

        def _zero(b, carry):
            pltpu.sync_copy(ps2.at[0], deg_sh.at[pl.ds(s * slab + b * CH, CH)])
            return carry

        lax.fori_loop(0, slab // CH, _zero, 0)
        plsc.subcore_barrier()

        def _issue(j):
            b = lax.rem(j, 2)
            pltpu.async_copy(pos8_h.at[sidx2.at[j]], ps2.at[b], sem_s.at[b])
            pltpu.async_copy(pos8_h.at[didx2.at[j]], pd2.at[b], sem_d.at[b])

        _issue(0)

        def _chunk(j, carry):
            b = lax.rem(j, 2)
            base = (wid * CPW + j) * CH

            @pl.when(j + 1 < CPW)
            def _():
                _issue(j + 1)

            pltpu.make_async_copy(pos8_h.at[sidx2.at[j]], ps2.at[b],
                                  sem_s.at[b]).wait()
            pltpu.make_async_copy(pos8_h.at[didx2.at[j]], pd2.at[b],
                                  sem_d.at[b]).wait()

            def _sub(r, carry2):
                ps2[b, r, :] = ps2[b, r, :] - pd2[b, r, :]
                return carry2

            lax.fori_loop(0, CH, _sub, 0)
            pltpu.sync_copy(ps2.at[b], rel8_h.at[pl.ds(base, CH)])
            pltpu.sync_copy(ones_v, deg_sh.at[didx2.at[j]], add=True)
            return carry

        lax.fori_loop(0, CPW, _chunk, 0)
        plsc.subcore_barrier()

        @pl.when(c == 0)
        def _():
            pltpu.sync_copy(deg_sh.at[pl.ds(s * slab, slab)],
                            deg0_h.at[pl.ds(s * slab, slab)])

        @pl.when(c == 1)
        def _():
            pltpu.sync_copy(deg_sh.at[pl.ds(s * slab, slab)],
                            deg1_h.at[pl.ds(s * slab, slab)])

    return body(pos8, src2, dst2)


def _stage_c(rel8, w1x, w1y, b1r, W2lo, W2hi, b2lo, b2hi, EP, D):
    BE2 = 1024
    EPH = EP // 2

    def half(rel, w1x_ref, w1y_ref, b1_ref, w2lo_ref, w2hi_ref,
             b2lo_ref, b2hi_ref):
        h = jnp.maximum(
            rel[:, 0:1] * w1x_ref[...] + rel[:, 1:2] * w1y_ref[...] + b1_ref[...],
            0.0).astype(jnp.bfloat16)
        lo = (jnp.dot(h, w2lo_ref[...], preferred_element_type=jnp.float32)
              + b2lo_ref[...]).astype(jnp.bfloat16)
        hi = (jnp.dot(h, w2hi_ref[...], preferred_element_type=jnp.float32)
              + b2hi_ref[...]).astype(jnp.bfloat16)
        return _pack_pairs(lo, hi)

    def body(relA_ref, relB_ref, w1x_ref, w1y_ref, b1_ref, w2lo_ref,
             w2hi_ref, b2lo_ref, b2hi_ref, ew_ref):
        pa = half(relA_ref[...], w1x_ref, w1y_ref, b1_ref, w2lo_ref,
                  w2hi_ref, b2lo_ref, b2hi_ref)
        pb = half(relB_ref[...], w1x_ref, w1y_ref, b1_ref, w2lo_ref,
                  w2hi_ref, b2lo_ref, b2hi_ref)
        ew_ref[...] = jnp.concatenate([pa, pb], axis=1)

    return pl.pallas_call(
        body,
        grid=(EPH // BE2,),
        in_specs=[
            pl.BlockSpec((BE2, 16), lambda i: (2 * i, 0)),
            pl.BlockSpec((BE2, 16), lambda i: (2 * i + 1, 0)),
            pl.BlockSpec((1, D), lambda i: (0, 0)),
            pl.BlockSpec((1, D), lambda i: (0, 0)),
            pl.BlockSpec((1, D), lambda i: (0, 0)),
            pl.BlockSpec((D, D // 2), lambda i: (0, 0)),
            pl.BlockSpec((D, D // 2), lambda i: (0, 0)),
            pl.BlockSpec((1, D // 2), lambda i: (0, 0)),
            pl.BlockSpec((1, D // 2), lambda i: (0, 0)),
        ],
        out_specs=pl.BlockSpec((BE2, D), lambda i: (i, 0)),
        out_shape=jax.ShapeDtypeStruct((EPH, D), jnp.float32),
    )(rel8, rel8, w1x, w1y, b1r, W2lo, W2hi, b2lo, b2hi)


def _stage_d(xw, srcA2, dstA2, srcB2, dstB2, ewp, NP, EPH, D):
    mesh = plsc.VectorSubcoreMesh(core_axis_name="c", subcore_axis_name="s")
    slab = NP // NS
    CHW = 32
    GS = 32
    WE = EPH // (NC * NS)
    NCH = WE // CHW
    NG = NCH // GS
    IRW = WE // 128
    IRG = IRW // NG

    @functools.partial(
        pl.kernel,
        out_type=(
            jax.ShapeDtypeStruct((NP, D), jnp.float32),
            jax.ShapeDtypeStruct((NP, D), jnp.float32),
        ),
        mesh=mesh,
        scratch_types=[
            pltpu.VMEM((2, IRG, 128), jnp.int32),
            pltpu.VMEM((2, IRG, 128), jnp.int32),
            pltpu.VMEM((2, IRG, 128), jnp.int32),
            pltpu.VMEM((2, IRG, 128), jnp.int32),
            pltpu.VMEM((2, CHW, D // 2), jnp.float32),
            pltpu.VMEM((2, CHW, D // 2), jnp.float32),
            pltpu.VMEM((2, CHW, D), jnp.float32),
            pltpu.VMEM((CHW, D), jnp.float32),
            pltpu.VMEM((CHW, D), jnp.float32),
            pltpu.VMEM_SHARED((NP, D), jnp.float32),
            pltpu.SemaphoreType.DMA((2,)),
            pltpu.SemaphoreType.DMA((2,)),
            pltpu.SemaphoreType.DMA((2,)),
            pltpu.SemaphoreType.DMA((2,)),
        ],
        compiler_params=pltpu.CompilerParams(use_tc_tiling_on_sc=False,
                                            needs_layout_passes=False),
    )
    def body(xw_h, srcA_h, dstA_h, srcB_h, dstB_h, ew_h, agg0_h, agg1_h,
             sidxA, didxA, sidxB, didxB, xsA2, xsB2, ew2, msgA, msgB,
             agg_sh, gsem, hsem, esem, isem):
        c = lax.axis_index("c")
        s = lax.axis_index("s")
        wid = c * NS + s

        def _zr(r, carry):
            for k in range(D // 16):
                msgA[r, pl.ds(k * 16, 16)] = jnp.zeros((16,), jnp.float32)
            return carry

        lax.fori_loop(0, CHW, _zr, 0)

        def _zero(b, carry):
            pltpu.sync_copy(msgA, agg_sh.at[pl.ds(s * slab + b * CHW, CHW)])
            return carry

        lax.fori_loop(0, slab // CHW, _zero, 0)
        plsc.subcore_barrier()

        def _idx_pairs(g, bg):
            row = wid * IRW + g * IRG
            return [
                (srcA_h.at[pl.ds(row, IRG)], sidxA.at[bg]),
                (dstA_h.at[pl.ds(row, IRG)], didxA.at[bg]),
                (srcB_h.at[pl.ds(row, IRG)], sidxB.at[bg]),
                (dstB_h.at[pl.ds(row, IRG)], didxB.at[bg]),
            ]

        def _issue_idx(g):
            bg = lax.rem(g, 2)
            for src, dst in _idx_pairs(g, bg):
                pltpu.async_copy(src, dst, isem.at[bg])

        def _wait_idx(g):
            bg = lax.rem(g, 2)
            for src, dst in _idx_pairs(g, bg):
                pltpu.make_async_copy(src, dst, isem.at[bg]).wait()

        def _slices(j):
            g = lax.div(j, GS)
            l = lax.rem(j, GS)
            return lax.rem(g, 2), lax.div(l, 4), lax.rem(l, 4) * CHW

        def _issue_gather(j):
            b = lax.rem(j, 2)
            bg, lr, lo = _slices(j)
            pr = wid * WE + j * CHW
            pltpu.async_copy(xw_h.at[sidxA.at[bg, lr, pl.ds(lo, CHW)]],
                             xsA2.at[b], gsem.at[b])
            pltpu.async_copy(xw_h.at[sidxB.at[bg, lr, pl.ds(lo, CHW)]],
                             xsB2.at[b], hsem.at[b])
            pltpu.async_copy(ew_h.at[pl.ds(pr, CHW)], ew2.at[b], esem.at[b])

        _issue_idx(0)
        _wait_idx(0)
        _issue_gather(0)

        def _chunk(j, carry):
            b = lax.rem(j, 2)
            g = lax.div(j, GS)
            bg, lr, lo = _slices(j)
            pr = wid * WE + j * CHW

            @pl.when(jnp.logical_and(lax.rem(j, GS) == 0, g + 1 < NG))
            def _():
                _issue_idx(g + 1)

            @pl.when(j + 1 < NCH)
            def _():
                @pl.when(lax.rem(j + 1, GS) == 0)
                def _():
                    _wait_idx(g + 1)

                _issue_gather(j + 1)

            pltpu.make_async_copy(xw_h.at[sidxA.at[bg, lr, pl.ds(lo, CHW)]],
                                  xsA2.at[b], gsem.at[b]).wait()
            pltpu.make_async_copy(xw_h.at[sidxB.at[bg, lr, pl.ds(lo, CHW)]],
                                  xsB2.at[b], hsem.at[b]).wait()
            pltpu.make_async_copy(ew_h.at[pl.ds(pr, CHW)], ew2.at[b],
                                  esem.at[b]).wait()

            def _mul(r, carry2):
                for k in range(D // 32):
                    slw = pl.ds(k * 16, 16)
                    slw2 = pl.ds(D // 2 + k * 16, 16)
                    sl0 = pl.ds(k * 32, 16)
                    sl1 = pl.ds(k * 32 + 16, 16)
                    xalo, xahi = plsc.unpack(
                        plsc.bitcast(xsA2[b, r, slw], jnp.bfloat16),
                        format=plsc.PackFormat.INTERLEAVED)
                    ealo, eahi = plsc.unpack(
                        plsc.bitcast(ew2[b, r, slw], jnp.bfloat16),
                        format=plsc.PackFormat.INTERLEAVED)
                    msgA[r, sl0] = xalo * ealo
                    msgA[r, sl1] = xahi * eahi
                    xblo, xbhi = plsc.unpack(
                        plsc.bitcast(xsB2[b, r, slw], jnp.bfloat16),
                        format=plsc.PackFormat.INTERLEAVED)
                    eblo, ebhi = plsc.unpack(
                        plsc.bitcast(ew2[b, r, slw2], jnp.bfloat16),
                        format=plsc.PackFormat.INTERLEAVED)
                    msgB[r, sl0] = xblo * eblo
                    msgB[r, sl1] = xbhi * ebhi
                return carry2

            lax.fori_loop(0, CHW, _mul, 0)
            pltpu.sync_copy(msgA,
                            agg_sh.at[didxA.at[bg, lr, pl.ds(lo, CHW)]],
                            add=True)
            pltpu.sync_copy(msgB,
                            agg_sh.at[didxB.at[bg, lr, pl.ds(lo, CHW)]],
                            add=True)
            return carry

        lax.fori_loop(0, NCH, _chunk, 0)
        plsc.subcore_barrier()

        @pl.when(c == 0)
        def _():
            pltpu.sync_copy(agg_sh.at[pl.ds(s * slab, slab)],
                            agg0_h.at[pl.ds(s * slab, slab)])

        @pl.when(c == 1)
        def _():
            pltpu.sync_copy(agg_sh.at[pl.ds(s * slab, slab)],
                            agg1_h.at[pl.ds(s * slab, slab)])

    return body(xw, srcA2, dstA2, srcB2, dstB2, ewp)


def _stage_e(pre, agg0, agg1, deg0, deg1, NP, D):
    BM = 1024

    def body(pre_ref, a0_ref, a1_ref, d0_ref, d1_ref, out_ref):
        deg = d0_ref[...][:, 0:1] + d1_ref[...][:, 0:1]
        r = 1.0 / jnp.maximum(deg, 1.0)
        out_ref[...] = jnp.maximum(
            pre_ref[...] + (a0_ref[...] + a1_ref[...]) * r, 0.0)

    return pl.pallas_call(
        body,
        grid=(NP // BM,),
        in_specs=[
            pl.BlockSpec((BM, D), lambda i: (i, 0)),
            pl.BlockSpec((BM, D), lambda i: (i, 0)),
            pl.BlockSpec((BM, D), lambda i: (i, 0)),
            pl.BlockSpec((BM, 16), lambda i: (i, 0)),
            pl.BlockSpec((BM, 16), lambda i: (i, 0)),
        ],
        out_specs=pl.BlockSpec((BM, D), lambda i: (i, 0)),
        out_shape=jax.ShapeDtypeStruct((NP, D), jnp.float32),
    )(pre, agg0, agg1, deg0, deg1)


def kernel(x, edge_index, pos, region, W_lin, W1, b1, W2, b2, W_root, bias,
           region_emb):
    N, D = x.shape
    E = edge_index.shape[1]
    NREG = region_emb.shape[0]

    NP = -(-(N + 1) // (NS * CH)) * (NS * CH)
    unit = NC * NS * 32 * 32
    EPH = -(-E // (2 * unit)) * unit
    EP = 2 * EPH
    CPWB = EP // (NC * NS * CH)
    BE2 = 1024

    srcf = jnp.pad(edge_index[0].astype(jnp.int32), (0, EP - E))
    dstf = jnp.pad(edge_index[1].astype(jnp.int32), (0, EP - E),
                   constant_values=N)
    sv = srcf.reshape(EP // BE2, BE2)
    dv = dstf.reshape(EP // BE2, BE2)
    srcA2 = sv[0::2].reshape(EPH // 128, 128)
    srcB2 = sv[1::2].reshape(EPH // 128, 128)
    dstA2 = dv[0::2].reshape(EPH // 128, 128)
    dstB2 = dv[1::2].reshape(EPH // 128, 128)
    pos8 = jnp.pad(pos, ((0, NP - N), (0, 16 - pos.shape[1])))
    xp = jnp.pad(x, ((0, NP - N), (0, 0)))
    regp = jnp.pad(region.astype(jnp.int32), (0, NP - N)).reshape(NP, 1)
    bias2 = bias.reshape(1, D)

    lo_idx = np.concatenate([np.arange(g * 32, g * 32 + 16)
                             for g in range(D // 32)])
    hi_idx = lo_idx + 16
    W_lo = W_lin[:, jnp.asarray(lo_idx)]
    W_hi = W_lin[:, jnp.asarray(hi_idx)]

    xw, pre = _stage_a(xp, W_lo, W_hi, W_root, regp, region_emb, bias2, NP, D,
                       NREG)
    rel8, deg0, deg1 = _stage_b(pos8, srcf.reshape(EP // CH, CH),
                                dstf.reshape(EP // CH, CH), NP, EP, CPWB)
    ewp = _stage_c(rel8,
                   W1[0].reshape(1, D), W1[1].reshape(1, D), b1.reshape(1, D),
                   W2[:, jnp.asarray(lo_idx)].astype(jnp.bfloat16),
                   W2[:, jnp.asarray(hi_idx)].astype(jnp.bfloat16),
                   b2[jnp.asarray(lo_idx)].reshape(1, D // 2),
                   b2[jnp.asarray(hi_idx)].reshape(1, D // 2), EP, D)
    agg0, agg1 = _stage_d(xw, srcA2, dstA2, srcB2, dstB2, ewp, NP, EPH, D)
    outp = _stage_e(pre, agg0, agg1, deg0, deg1, NP, D)
    return outp[:N]


# speedup vs baseline: 1.2984x; 1.0000x over previous
import functools

import jax
import jax.numpy as jnp
import numpy as np
from jax import lax
from jax.experimental import pallas as pl
from jax.experimental.pallas import tpu as pltpu
from jax.experimental.pallas import tpu_sc as plsc

NC = 2
NS = 16
CH = 128


def _pack_pairs(lo_bf, hi_bf):
    lo = lax.bitcast_convert_type(lo_bf, jnp.uint16).astype(jnp.uint32)
    hi = lax.bitcast_convert_type(hi_bf, jnp.uint16).astype(jnp.uint32)
    return lax.bitcast_convert_type(lo | (hi << 16), jnp.float32)


def _stage_a(xp, W_lo, W_hi, W_root, regp, region_emb, bias2, NP, D, NREG):
    BM = 1024

    def body(x_ref, wlo_ref, whi_ref, wr_ref, reg_ref, emb_ref, b_ref,
             xw_ref, pre_ref):
        xb = x_ref[...]
        lo = jnp.dot(xb, wlo_ref[...],
                     preferred_element_type=jnp.float32).astype(jnp.bfloat16)
        hi = jnp.dot(xb, whi_ref[...],
                     preferred_element_type=jnp.float32).astype(jnp.bfloat16)
        xw_ref[...] = _pack_pairs(lo, hi)
        oh = (reg_ref[...] == lax.broadcasted_iota(jnp.int32, (BM, NREG), 1))
        pre_ref[...] = (
            jnp.dot(xb, wr_ref[...], preferred_element_type=jnp.float32)
            + jnp.dot(oh.astype(jnp.float32), emb_ref[...],
                      preferred_element_type=jnp.float32)
            + b_ref[...]
        )

    return pl.pallas_call(
        body,
        grid=(NP // BM,),
        in_specs=[
            pl.BlockSpec((BM, D), lambda i: (i, 0)),
            pl.BlockSpec((D, D // 2), lambda i: (0, 0)),
            pl.BlockSpec((D, D // 2), lambda i: (0, 0)),
            pl.BlockSpec((D, D), lambda i: (0, 0)),
            pl.BlockSpec((BM, 1), lambda i: (i, 0)),
            pl.BlockSpec((NREG, D), lambda i: (0, 0)),
            pl.BlockSpec((1, D), lambda i: (0, 0)),
        ],
        out_specs=[pl.BlockSpec((BM, D // 2), lambda i: (i, 0)),
                   pl.BlockSpec((BM, D), lambda i: (i, 0))],
        out_shape=[jax.ShapeDtypeStruct((NP, D // 2), jnp.float32),
                   jax.ShapeDtypeStruct((NP, D), jnp.float32)],
    )(xp, W_lo, W_hi, W_root, regp, region_emb, bias2)


def _stage_b(pos8, src2, dst2, NP, EP, CPW):
    mesh = plsc.VectorSubcoreMesh(core_axis_name="c", subcore_axis_name="s")
    slab = NP // NS

    @functools.partial(
        pl.kernel,
        out_type=(
            jax.ShapeDtypeStruct((EP, 16), jnp.float32),
            jax.ShapeDtypeStruct((NP, 16), jnp.float32),
            jax.ShapeDtypeStruct((NP, 16), jnp.float32),
        ),
        mesh=mesh,
        scratch_types=[
            pltpu.VMEM((CPW, CH), jnp.int32),
            pltpu.VMEM((CPW, CH), jnp.int32),
            pltpu.VMEM((2, CH, 16), jnp.float32),
            pltpu.VMEM((2, CH, 16), jnp.float32),
            pltpu.VMEM((CH, 16), jnp.float32),
            pltpu.VMEM_SHARED((NP, 16), jnp.float32),
            pltpu.SemaphoreType.DMA((2,)),
            pltpu.SemaphoreType.DMA((2,)),
        ],
        compiler_params=pltpu.CompilerParams(use_tc_tiling_on_sc=False,
                                            needs_layout_passes=False),
    )
    def body(pos8_h, src_h, dst_h, rel8_h, deg0_h, deg1_h,
             sidx2, didx2, ps2, pd2, ones_v, deg_sh, sem_s, sem_d):
        c = lax.axis_index("c")
        s = lax.axis_index("s")
        wid = c * NS + s
        pltpu.sync_copy(src_h.at[pl.ds(wid * CPW, CPW)], sidx2)
        pltpu.sync_copy(dst_h.at[pl.ds(wid * CPW, CPW)], didx2)

        def _fill(i, carry):
            ones_v[i, :] = jnp.ones((16,), jnp.float32)
            ps2[0, i, :] = jnp.zeros((16,), jnp.float32)
            return carry

        lax.fori_loop(0, CH, _fill, 0)

        def _zero(b, carry):
            pltpu.sync_copy(ps2.at[0], deg_sh.at[pl.ds(s * slab + b * CH, CH)])
            return carry

        lax.fori_loop(0, slab // CH, _zero, 0)
        plsc.subcore_barrier()

        def _issue(j):
            b = lax.rem(j, 2)
            pltpu.async_copy(pos8_h.at[sidx2.at[j]], ps2.at[b], sem_s.at[b])
            pltpu.async_copy(pos8_h.at[didx2.at[j]], pd2.at[b], sem_d.at[b])

        _issue(0)

        def _chunk(j, carry):
            b = lax.rem(j, 2)
            base = (wid * CPW + j) * CH

            @pl.when(j + 1 < CPW)
            def _():
                _issue(j + 1)

            pltpu.make_async_copy(pos8_h.at[sidx2.at[j]], ps2.at[b],
                                  sem_s.at[b]).wait()
            pltpu.make_async_copy(pos8_h.at[didx2.at[j]], pd2.at[b],
                                  sem_d.at[b]).wait()

            def _sub(r, carry2):
                ps2[b, r, :] = ps2[b, r, :] - pd2[b, r, :]
                return carry2

            lax.fori_loop(0, CH, _sub, 0)
            pltpu.sync_copy(ps2.at[b], rel8_h.at[pl.ds(base, CH)])
            pltpu.sync_copy(ones_v, deg_sh.at[didx2.at[j]], add=True)
            return carry

        lax.fori_loop(0, CPW, _chunk, 0)
        plsc.subcore_barrier()

        @pl.when(c == 0)
        def _():
            pltpu.sync_copy(deg_sh.at[pl.ds(s * slab, slab)],
                            deg0_h.at[pl.ds(s * slab, slab)])

        @pl.when(c == 1)
        def _():
            pltpu.sync_copy(deg_sh.at[pl.ds(s * slab, slab)],
                            deg1_h.at[pl.ds(s * slab, slab)])

    return body(pos8, src2, dst2)


def _stage_c(rel8, w1x, w1y, b1r, W2lo, W2hi, b2lo, b2hi, EP, D):
    BE2 = 1024
    EPH = EP // 2

    def half(rel, w1x_ref, w1y_ref, b1_ref, w2lo_ref, w2hi_ref,
             b2lo_ref, b2hi_ref):
        h = jnp.maximum(
            rel[:, 0:1] * w1x_ref[...] + rel[:, 1:2] * w1y_ref[...] + b1_ref[...],
            0.0).astype(jnp.bfloat16)
        lo = (jnp.dot(h, w2lo_ref[...], preferred_element_type=jnp.float32)
              + b2lo_ref[...]).astype(jnp.bfloat16)
        hi = (jnp.dot(h, w2hi_ref[...], preferred_element_type=jnp.float32)
              + b2hi_ref[...]).astype(jnp.bfloat16)
        return _pack_pairs(lo, hi)

    def body(relA_ref, relB_ref, w1x_ref, w1y_ref, b1_ref, w2lo_ref,
             w2hi_ref, b2lo_ref, b2hi_ref, ew_ref):
        pa = half(relA_ref[...], w1x_ref, w1y_ref, b1_ref, w2lo_ref,
                  w2hi_ref, b2lo_ref, b2hi_ref)
        pb = half(relB_ref[...], w1x_ref, w1y_ref, b1_ref, w2lo_ref,
                  w2hi_ref, b2lo_ref, b2hi_ref)
        ew_ref[...] = jnp.concatenate([pa, pb], axis=1)

    return pl.pallas_call(
        body,
        grid=(EPH // BE2,),
        in_specs=[
            pl.BlockSpec((BE2, 16), lambda i: (2 * i, 0)),
            pl.BlockSpec((BE2, 16), lambda i: (2 * i + 1, 0)),
            pl.BlockSpec((1, D), lambda i: (0, 0)),
            pl.BlockSpec((1, D), lambda i: (0, 0)),
            pl.BlockSpec((1, D), lambda i: (0, 0)),
            pl.BlockSpec((D, D // 2), lambda i: (0, 0)),
            pl.BlockSpec((D, D // 2), lambda i: (0, 0)),
            pl.BlockSpec((1, D // 2), lambda i: (0, 0)),
            pl.BlockSpec((1, D // 2), lambda i: (0, 0)),
        ],
        out_specs=pl.BlockSpec((BE2, D), lambda i: (i, 0)),
        out_shape=jax.ShapeDtypeStruct((EPH, D), jnp.float32),
    )(rel8, rel8, w1x, w1y, b1r, W2lo, W2hi, b2lo, b2hi)


def _stage_d(xw, srcA2, dstA2, srcB2, dstB2, ewp, NP, EPH, D):
    mesh = plsc.VectorSubcoreMesh(core_axis_name="c", subcore_axis_name="s")
    slab = NP // NS
    CHW = 32
    GS = 32
    WE = EPH // (NC * NS)
    NCH = WE // CHW
    NG = NCH // GS
    IRW = WE // 128
    IRG = IRW // NG

    @functools.partial(
        pl.kernel,
        out_type=(
            jax.ShapeDtypeStruct((NP, D), jnp.float32),
            jax.ShapeDtypeStruct((NP, D), jnp.float32),
        ),
        mesh=mesh,
        scratch_types=[
            pltpu.VMEM((2, IRG, 128), jnp.int32),
            pltpu.VMEM((2, IRG, 128), jnp.int32),
            pltpu.VMEM((2, IRG, 128), jnp.int32),
            pltpu.VMEM((2, IRG, 128), jnp.int32),
            pltpu.VMEM((2, CHW, D // 2), jnp.float32),
            pltpu.VMEM((2, CHW, D // 2), jnp.float32),
            pltpu.VMEM((2, CHW, D), jnp.float32),
            pltpu.VMEM((CHW, D), jnp.float32),
            pltpu.VMEM((CHW, D), jnp.float32),
            pltpu.VMEM_SHARED((NP, D), jnp.float32),
            pltpu.SemaphoreType.DMA((2,)),
            pltpu.SemaphoreType.DMA((2,)),
            pltpu.SemaphoreType.DMA((2,)),
            pltpu.SemaphoreType.DMA((2,)),
        ],
        compiler_params=pltpu.CompilerParams(use_tc_tiling_on_sc=False,
                                            needs_layout_passes=False),
    )
    def body(xw_h, srcA_h, dstA_h, srcB_h, dstB_h, ew_h, agg0_h, agg1_h,
             sidxA, didxA, sidxB, didxB, xsA2, xsB2, ew2, msgA, msgB,
             agg_sh, gsem, hsem, esem, isem):
        c = lax.axis_index("c")
        s = lax.axis_index("s")
        wid = c * NS + s

        def _zr(r, carry):
            for k in range(D // 16):
                msgA[r, pl.ds(k * 16, 16)] = jnp.zeros((16,), jnp.float32)
            return carry

        lax.fori_loop(0, CHW, _zr, 0)

        def _zero(b, carry):
            pltpu.sync_copy(msgA, agg_sh.at[pl.ds(s * slab + b * CHW, CHW)])
            return carry

        lax.fori_loop(0, slab // CHW, _zero, 0)
        plsc.subcore_barrier()

        def _idx_pairs(g, bg):
            row = wid * IRW + g * IRG
            return [
                (srcA_h.at[pl.ds(row, IRG)], sidxA.at[bg]),
                (dstA_h.at[pl.ds(row, IRG)], didxA.at[bg]),
                (srcB_h.at[pl.ds(row, IRG)], sidxB.at[bg]),
                (dstB_h.at[pl.ds(row, IRG)], didxB.at[bg]),
            ]

        def _issue_idx(g):
            bg = lax.rem(g, 2)
            for src, dst in _idx_pairs(g, bg):
                pltpu.async_copy(src, dst, isem.at[bg])

        def _wait_idx(g):
            bg = lax.rem(g, 2)
            for src, dst in _idx_pairs(g, bg):
                pltpu.make_async_copy(src, dst, isem.at[bg]).wait()

        def _slices(j):
            g = lax.div(j, GS)
            l = lax.rem(j, GS)
            return lax.rem(g, 2), lax.div(l, 4), lax.rem(l, 4) * CHW

        def _issue_gather(j):
            b = lax.rem(j, 2)
            bg, lr, lo = _slices(j)
            pr = wid * WE + j * CHW
            pltpu.async_copy(xw_h.at[sidxA.at[bg, lr, pl.ds(lo, CHW)]],
                             xsA2.at[b], gsem.at[b])
            pltpu.async_copy(xw_h.at[sidxB.at[bg, lr, pl.ds(lo, CHW)]],
                             xsB2.at[b], hsem.at[b])
            pltpu.async_copy(ew_h.at[pl.ds(pr, CHW)], ew2.at[b], esem.at[b])

        _issue_idx(0)
        _wait_idx(0)
        _issue_gather(0)

        def _chunk(j, carry):
            b = lax.rem(j, 2)
            g = lax.div(j, GS)
            bg, lr, lo = _slices(j)
            pr = wid * WE + j * CHW

            @pl.when(jnp.logical_and(lax.rem(j, GS) == 0, g + 1 < NG))
            def _():
                _issue_idx(g + 1)

            @pl.when(j + 1 < NCH)
            def _():
                @pl.when(lax.rem(j + 1, GS) == 0)
                def _():
                    _wait_idx(g + 1)

                _issue_gather(j + 1)

            pltpu.make_async_copy(xw_h.at[sidxA.at[bg, lr, pl.ds(lo, CHW)]],
                                  xsA2.at[b], gsem.at[b]).wait()
            pltpu.make_async_copy(xw_h.at[sidxB.at[bg, lr, pl.ds(lo, CHW)]],
                                  xsB2.at[b], hsem.at[b]).wait()
            pltpu.make_async_copy(ew_h.at[pl.ds(pr, CHW)], ew2.at[b],
                                  esem.at[b]).wait()

            def _mul(r, carry2):
                for k in range(D // 32):
                    slw = pl.ds(k * 16, 16)
                    slw2 = pl.ds(D // 2 + k * 16, 16)
                    sl0 = pl.ds(k * 32, 16)
                    sl1 = pl.ds(k * 32 + 16, 16)
                    xalo, xahi = plsc.unpack(
                        plsc.bitcast(xsA2[b, r, slw], jnp.bfloat16),
                        format=plsc.PackFormat.INTERLEAVED)
                    ealo, eahi = plsc.unpack(
                        plsc.bitcast(ew2[b, r, slw], jnp.bfloat16),
                        format=plsc.PackFormat.INTERLEAVED)
                    msgA[r, sl0] = xalo * ealo
                    msgA[r, sl1] = xahi * eahi
                    xblo, xbhi = plsc.unpack(
                        plsc.bitcast(xsB2[b, r, slw], jnp.bfloat16),
                        format=plsc.PackFormat.INTERLEAVED)
                    eblo, ebhi = plsc.unpack(
                        plsc.bitcast(ew2[b, r, slw2], jnp.bfloat16),
                        format=plsc.PackFormat.INTERLEAVED)
                    msgB[r, sl0] = xblo * eblo
                    msgB[r, sl1] = xbhi * ebhi
                return carry2

            lax.fori_loop(0, CHW, _mul, 0)
            pltpu.sync_copy(msgA,
                            agg_sh.at[didxA.at[bg, lr, pl.ds(lo, CHW)]],
                            add=True)
            pltpu.sync_copy(msgB,
                            agg_sh.at[didxB.at[bg, lr, pl.ds(lo, CHW)]],
                            add=True)
            return carry

        lax.fori_loop(0, NCH, _chunk, 0)
        plsc.subcore_barrier()

        @pl.when(c == 0)
        def _():
            pltpu.sync_copy(agg_sh.at[pl.ds(s * slab, slab)],
                            agg0_h.at[pl.ds(s * slab, slab)])

        @pl.when(c == 1)
        def _():
            pltpu.sync_copy(agg_sh.at[pl.ds(s * slab, slab)],
                            agg1_h.at[pl.ds(s * slab, slab)])

    return body(xw, srcA2, dstA2, srcB2, dstB2, ewp)


def _stage_e(pre, agg0, agg1, deg0, deg1, NP, D):
    BM = 1024

    def body(pre_ref, a0_ref, a1_ref, d0_ref, d1_ref, out_ref):
        deg = d0_ref[...][:, 0:1] + d1_ref[...][:, 0:1]
        r = 1.0 / jnp.maximum(deg, 1.0)
        out_ref[...] = jnp.maximum(
            pre_ref[...] + (a0_ref[...] + a1_ref[...]) * r, 0.0)

    return pl.pallas_call(
        body,
        grid=(NP // BM,),
        in_specs=[
            pl.BlockSpec((BM, D), lambda i: (i, 0)),
            pl.BlockSpec((BM, D), lambda i: (i, 0)),
            pl.BlockSpec((BM, D), lambda i: (i, 0)),
            pl.BlockSpec((BM, 16), lambda i: (i, 0)),
            pl.BlockSpec((BM, 16), lambda i: (i, 0)),
        ],
        out_specs=pl.BlockSpec((BM, D), lambda i: (i, 0)),
        out_shape=jax.ShapeDtypeStruct((NP, D), jnp.float32),
    )(pre, agg0, agg1, deg0, deg1)


def kernel(x, edge_index, pos, region, W_lin, W1, b1, W2, b2, W_root, bias,
           region_emb):
    N, D = x.shape
    E = edge_index.shape[1]
    NREG = region_emb.shape[0]

    NP = -(-(N + 1) // (NS * CH)) * (NS * CH)
    unit = NC * NS * 32 * 32
    EPH = -(-E // (2 * unit)) * unit
    EP = 2 * EPH
    CPWB = EP // (NC * NS * CH)
    BE2 = 1024

    srcf = jnp.pad(edge_index[0].astype(jnp.int32), (0, EP - E))
    dstf = jnp.pad(edge_index[1].astype(jnp.int32), (0, EP - E),
                   constant_values=N)
    sv = srcf.reshape(EP // BE2, BE2)
    dv = dstf.reshape(EP // BE2, BE2)
    srcA2 = sv[0::2].reshape(EPH // 128, 128)
    srcB2 = sv[1::2].reshape(EPH // 128, 128)
    dstA2 = dv[0::2].reshape(EPH // 128, 128)
    dstB2 = dv[1::2].reshape(EPH // 128, 128)
    pos8 = jnp.pad(pos, ((0, NP - N), (0, 16 - pos.shape[1])))
    xp = jnp.pad(x, ((0, NP - N), (0, 0)))
    regp = jnp.pad(region.astype(jnp.int32), (0, NP - N)).reshape(NP, 1)
    bias2 = bias.reshape(1, D)

    lo_idx = np.concatenate([np.arange(g * 32, g * 32 + 16)
                             for g in range(D // 32)])
    hi_idx = lo_idx + 16
    W_lo = W_lin[:, jnp.asarray(lo_idx)]
    W_hi = W_lin[:, jnp.asarray(hi_idx)]

    xw, pre = _stage_a(xp, W_lo, W_hi, W_root, regp, region_emb, bias2, NP, D,
                       NREG)
    rel8, deg0, deg1 = _stage_b(pos8, srcf.reshape(EP // CH, CH),
                                dstf.reshape(EP // CH, CH), NP, EP, CPWB)
    ewp = _stage_c(rel8,
                   W1[0].reshape(1, D), W1[1].reshape(1, D), b1.reshape(1, D),
                   W2[:, jnp.asarray(lo_idx)].astype(jnp.bfloat16),
                   W2[:, jnp.asarray(hi_idx)].astype(jnp.bfloat16),
                   b2[jnp.asarray(lo_idx)].reshape(1, D // 2),
                   b2[jnp.asarray(hi_idx)].reshape(1, D // 2), EP, D)
    agg0, agg1 = _stage_d(xw, srcA2, dstA2, srcB2, dstB2, ewp, NP, EPH, D)
    outp = _stage_e(pre, agg0, agg1, deg0, deg1, NP, D)
    return outp[:N]
